# baseline (speedup 1.0000x reference)
import dataclasses
import functools

import jax
import jax.numpy as jnp
from jax import lax
from jax.experimental import pallas as pl
from jax.experimental.pallas import tpu as pltpu
from jax.experimental.pallas import tpu_sc as plsc

B, N, FE, M, H, K = 4, 4096, 16, 3072, 64, 64
TM = 256
TR = 256
BM = B * M


def _topk_body(pix_ref, los_ref, vals_ref, idx_ref, gidxt_ref):
    b = pl.program_id(0)
    cos = jnp.dot(pix_ref[...], los_ref[0],
                  preferred_element_type=jnp.float32)
    cos = jnp.clip(cos, -1.0, 1.0)
    iota = lax.broadcasted_iota(jnp.int32, (TM, N), 1)
    lanek = lax.broadcasted_iota(jnp.int32, (TM, K), 1)

    def body(k, carry):
        work, ov, oi = carry
        mx = jnp.max(work, axis=1, keepdims=True)
        am = jnp.min(jnp.where(work == mx, iota, N), axis=1,
                     keepdims=True)
        ov = jnp.where(lanek == k, mx, ov)
        oi = jnp.where(lanek == k, am, oi)
        work = jnp.where(iota == am, -2.0, work)
        return work, ov, oi

    init = (cos,
            jnp.zeros((TM, K), jnp.float32),
            jnp.zeros((TM, K), jnp.int32))
    _, ov, oi = lax.fori_loop(0, K, body, init)
    vals_ref[0] = ov
    idx_ref[0] = oi
    gidxt_ref[...] = jnp.transpose(oi + b * N, (1, 0))


def _run_topk(r_pix, r_losT, interpret=False):
    grid = (B, M // TM)
    return pl.pallas_call(
        _topk_body,
        grid=grid,
        in_specs=[
            pl.BlockSpec((TM, 3), lambda b, mt: (mt, 0)),
            pl.BlockSpec((1, 3, N), lambda b, mt: (b, 0, 0)),
        ],
        out_specs=[
            pl.BlockSpec((1, TM, K), lambda b, mt: (b, mt, 0)),
            pl.BlockSpec((1, TM, K), lambda b, mt: (b, mt, 0)),
            pl.BlockSpec((K, TM), lambda b, mt: (0, b * (M // TM) + mt)),
        ],
        out_shape=[
            jax.ShapeDtypeStruct((B, M, K), jnp.float32),
            jax.ShapeDtypeStruct((B, M, K), jnp.int32),
            jax.ShapeDtypeStruct((K, BM), jnp.int32),
        ],
        interpret=interpret,
    )(r_pix, r_losT)


def _sc_gather(x_pad, gidx_flat):
    num_idx = gidx_flat.shape[0]
    window = 128
    lanes = x_pad.shape[1]
    idx2d = gidx_flat.reshape(1, num_idx)
    mesh = plsc.VectorSubcoreMesh(core_axis_name="core",
                                  subcore_axis_name="subcore")

    @functools.partial(
        pl.kernel,
        out_type=jax.ShapeDtypeStruct((num_idx, lanes), jnp.float32),
        mesh=mesh)
    def gather_kernel(x_hbm, i_hbm, o_hbm):
        def body(i_vmem, o_vmem):
            pltpu.sync_copy(x_hbm.at[i_vmem.at[0]], o_vmem)

        pltpu.emit_pipeline(
            body,
            grid=(num_idx // window,),
            in_specs=[pl.BlockSpec((1, window), index_map=lambda i: (0, i))],
            out_specs=[pl.BlockSpec((window, lanes),
                                    index_map=lambda i: (i, 0))],
            core_axis_name=("core", "subcore"),
            dimension_semantics=(pltpu.PARALLEL,),
        )(i_hbm, o_hbm)

    return gather_kernel(x_pad, idx2d)


def _sc_gather2(x128, gidxt):
    CH = 256
    XROWS = N * FE // 128
    mesh = plsc.VectorSubcoreMesh(core_axis_name="core",
                                  subcore_axis_name="subcore")
    cp = pltpu.CompilerParams()
    if "needs_layout_passes" in pltpu.CompilerParams.__dataclass_fields__:
        cp = dataclasses.replace(cp, needs_layout_passes=False)

    @functools.partial(
        pl.kernel,
        out_type=jax.ShapeDtypeStruct((K * BM, FE), jnp.float32),
        mesh=mesh,
        compiler_params=cp,
        scratch_types=[
            pltpu.VMEM((XROWS, 128), jnp.float32),
            pltpu.VMEM((M,), jnp.int32),
            pltpu.VMEM((CH, FE), jnp.float32),
        ])
    def gk(x_hbm, i_hbm, o_hbm, xb, ib, chb):
        wid = lax.axis_index("core") * 16 + lax.axis_index("subcore")
        iota16 = lax.broadcasted_iota(jnp.int32, (16,), 0)

        @pl.loop(0, B)
        def _b(b):
            pltpu.sync_copy(x_hbm.at[pl.ds(b * XROWS, XROWS)], xb)

            @pl.loop(0, 2)
            def _kk(kk):
                k = wid * 2 + kk
                pltpu.sync_copy(i_hbm.at[pl.ds(k * BM + b * M, M)], ib)

                @pl.loop(0, M // CH)
                def _c(c):
                    for g in range(CH // 16):
                        nloc = ib[pl.ds(c * CH + g * 16, 16)] - b * N
                        rowv = lax.shift_right_logical(nloc, 3)
                        colb = (nloc & 7) * FE
                        orow = iota16 + g * 16
                        for col in range(FE):
                            v = plsc.load_gather(
                                xb, [rowv, colb + col])
                            plsc.store_scatter(
                                chb, [orow, jnp.full((16,), col, jnp.int32)],
                                v)
                    base = k * BM + b * M + c * CH
                    pltpu.sync_copy(chb, o_hbm.at[pl.ds(base, CH)])

    return gk(x128, gidxt)


def _arccos(v):
    a = jnp.abs(v)
    p = jnp.float32(-0.0012624911)
    p = p * a + jnp.float32(0.0066700901)
    p = p * a + jnp.float32(-0.0170881256)
    p = p * a + jnp.float32(0.0308918810)
    p = p * a + jnp.float32(-0.0501743046)
    p = p * a + jnp.float32(0.0889789874)
    p = p * a + jnp.float32(-0.2145988016)
    p = p * a + jnp.float32(1.5707963050)
    r = jnp.sqrt(jnp.maximum(1.0 - a, 0.0)) * p
    return jnp.where(v < 0, jnp.float32(3.1415927410125732) - r, r)


def _mlp_body(xgt_ref, vals_ref, w1a_ref, w1d_ref, b1_ref, w2_ref, b2_ref,
              out_ref, lg_ref):
    vals = vals_ref[...]
    d = _arccos(jnp.clip(vals, -1.0, 1.0))
    w1a = w1a_ref[...]
    w1d = w1d_ref[...]
    b1v = b1_ref[...]
    w2c = w2_ref[...]
    for k in range(K):
        xk = xgt_ref[k][:, :FE]
        a1k = jnp.dot(xk, w1a, preferred_element_type=jnp.float32)
        hk = jnp.maximum(a1k + d[:, k:k + 1] * w1d + b1v, 0.0)
        lg_ref[:, k:k + 1] = jnp.dot(hk, w2c,
                                     preferred_element_type=jnp.float32)
    lg = lg_ref[...] + b2_ref[0, 0]
    mx = jnp.max(lg, axis=1, keepdims=True)
    e = jnp.exp(lg - mx)
    w = e / jnp.sum(e, axis=1, keepdims=True)
    acc = jnp.zeros((TR, FE), jnp.float32)
    for k in range(K):
        acc = acc + w[:, k:k + 1] * xgt_ref[k][:, :FE]
    out_ref[...] = acc


def _run_mlp(xgt, vals2d, W1a, w1d, b1, W2r, b2, interpret=False):
    grid = (BM // TR,)
    return pl.pallas_call(
        _mlp_body,
        grid=grid,
        in_specs=[
            pl.BlockSpec((K, TR, FE), lambda r: (0, r, 0)),
            pl.BlockSpec((TR, K), lambda r: (r, 0)),
            pl.BlockSpec((FE, H), lambda r: (0, 0)),
            pl.BlockSpec((1, H), lambda r: (0, 0)),
            pl.BlockSpec((1, H), lambda r: (0, 0)),
            pl.BlockSpec((H, 1), lambda r: (0, 0)),
            pl.BlockSpec((1, 1), lambda r: (0, 0)),
        ],
        out_specs=pl.BlockSpec((TR, FE), lambda r: (r, 0)),
        out_shape=jax.ShapeDtypeStruct((BM, FE), jnp.float32),
        scratch_shapes=[pltpu.VMEM((TR, K), jnp.float32)],
        interpret=interpret,
    )(xgt, vals2d, W1a, w1d, b1, W2r, b2)


def _sph_to_cart(theta, phi):
    st = jnp.sin(theta)
    return jnp.stack([st * jnp.cos(phi), st * jnp.sin(phi),
                      jnp.cos(theta)], axis=-1)


def kernel(x, los_theta_phi, pix_theta_phi, W1, b1, W2, b2):
    r_los = _sph_to_cart(los_theta_phi[..., 0], los_theta_phi[..., 1])
    r_pix = _sph_to_cart(pix_theta_phi[:, 0], pix_theta_phi[:, 1])
    r_losT = jnp.transpose(r_los, (0, 2, 1))

    vals, idx, gidxt = _run_topk(r_pix, r_losT)

    xg = _sc_gather2(x.reshape(B * N * FE // 128, 128),
                     gidxt.reshape(K * BM))
    xgt = xg.reshape(K, BM, FE)

    W1a = W1[:FE, :]
    w1d = W1[FE:FE + 1, :]
    pooled = _run_mlp(xgt, vals.reshape(BM, K), W1a, w1d,
                      b1.reshape(1, H), W2, b2.reshape(1, 1))
    return pooled.reshape(B, M, FE), idx


# speedup vs baseline: 1.7345x; 1.7345x over previous
import dataclasses
import functools

import jax
import jax.numpy as jnp
from jax import lax
from jax.experimental import pallas as pl
from jax.experimental.pallas import tpu as pltpu
from jax.experimental.pallas import tpu_sc as plsc

B, N, FE, M, H, K = 4, 4096, 16, 3072, 64, 64
TM = 256
TR = 256
BM = B * M


def _topk_body(pix_ref, los_ref, vals_ref, idx_ref, gidxt_ref, wk_ref):
    b = pl.program_id(0)
    cos = jnp.dot(pix_ref[...], los_ref[0],
                  preferred_element_type=jnp.float32)
    wk_ref[...] = jnp.clip(cos, -1.0, 1.0)
    iota = lax.broadcasted_iota(jnp.int32, (TM, N), 1)
    lanek = lax.broadcasted_iota(jnp.int32, (TM, K), 1)

    def body(k, carry):
        ov, oi = carry
        mx = jnp.max(wk_ref[...], axis=1, keepdims=True)
        am = jnp.min(jnp.where(wk_ref[...] == mx, iota, N), axis=1,
                     keepdims=True)
        ov = jnp.where(lanek == k, mx, ov)
        oi = jnp.where(lanek == k, am, oi)
        wk_ref[...] = jnp.where(iota == am, -2.0, wk_ref[...])
        return ov, oi

    init = (jnp.zeros((TM, K), jnp.float32),
            jnp.zeros((TM, K), jnp.int32))
    ov, oi = lax.fori_loop(0, K, body, init)
    vals_ref[...] = jnp.transpose(ov, (1, 0))
    idx_ref[0] = oi
    gidxt_ref[...] = jnp.transpose(oi + b * N, (1, 0))


def _run_topk(r_pix, r_losT, interpret=False):
    grid = (B, M // TM)
    return pl.pallas_call(
        _topk_body,
        grid=grid,
        in_specs=[
            pl.BlockSpec((TM, 3), lambda b, mt: (mt, 0)),
            pl.BlockSpec((1, 3, N), lambda b, mt: (b, 0, 0)),
        ],
        out_specs=[
            pl.BlockSpec((K, TM), lambda b, mt: (0, b * (M // TM) + mt)),
            pl.BlockSpec((1, TM, K), lambda b, mt: (b, mt, 0)),
            pl.BlockSpec((K, TM), lambda b, mt: (0, b * (M // TM) + mt)),
        ],
        out_shape=[
            jax.ShapeDtypeStruct((K, BM), jnp.float32),
            jax.ShapeDtypeStruct((B, M, K), jnp.int32),
            jax.ShapeDtypeStruct((K, BM), jnp.int32),
        ],
        scratch_shapes=[pltpu.VMEM((TM, N), jnp.float32)],
        interpret=interpret,
    )(r_pix, r_losT)


def _sc_gather(x_pad, gidx_flat):
    num_idx = gidx_flat.shape[0]
    window = 128
    lanes = x_pad.shape[1]
    idx2d = gidx_flat.reshape(1, num_idx)
    mesh = plsc.VectorSubcoreMesh(core_axis_name="core",
                                  subcore_axis_name="subcore")

    @functools.partial(
        pl.kernel,
        out_type=jax.ShapeDtypeStruct((num_idx, lanes), jnp.float32),
        mesh=mesh)
    def gather_kernel(x_hbm, i_hbm, o_hbm):
        def body(i_vmem, o_vmem):
            pltpu.sync_copy(x_hbm.at[i_vmem.at[0]], o_vmem)

        pltpu.emit_pipeline(
            body,
            grid=(num_idx // window,),
            in_specs=[pl.BlockSpec((1, window), index_map=lambda i: (0, i))],
            out_specs=[pl.BlockSpec((window, lanes),
                                    index_map=lambda i: (i, 0))],
            core_axis_name=("core", "subcore"),
            dimension_semantics=(pltpu.PARALLEL,),
        )(i_hbm, o_hbm)

    return gather_kernel(x_pad, idx2d)


def _sc_gather2(x128, gidxt):
    CH = 256
    XROWS = N * FE // 128
    mesh = plsc.VectorSubcoreMesh(core_axis_name="core",
                                  subcore_axis_name="subcore")
    cp = pltpu.CompilerParams()
    if "needs_layout_passes" in pltpu.CompilerParams.__dataclass_fields__:
        cp = dataclasses.replace(cp, needs_layout_passes=False)

    @functools.partial(
        pl.kernel,
        out_type=jax.ShapeDtypeStruct((FE, K * BM), jnp.float32),
        mesh=mesh,
        compiler_params=cp,
        scratch_types=[
            pltpu.VMEM((XROWS, 128), jnp.float32),
            pltpu.VMEM((M,), jnp.int32),
            pltpu.VMEM((FE, CH), jnp.float32),
        ])
    def gk(x_hbm, i_hbm, o_hbm, xb, ib, chb):
        wid = lax.axis_index("core") * 16 + lax.axis_index("subcore")
        iota16 = lax.broadcasted_iota(jnp.int32, (16,), 0)

        @pl.loop(0, B)
        def _b(b):
            pltpu.sync_copy(x_hbm.at[pl.ds(b * XROWS, XROWS)], xb)

            @pl.loop(0, 2)
            def _kk(kk):
                k = wid * 2 + kk
                pltpu.sync_copy(i_hbm.at[pl.ds(k * BM + b * M, M)], ib)

                @pl.loop(0, M // CH)
                def _c(c):
                    for g in range(CH // 16):
                        nloc = ib[pl.ds(c * CH + g * 16, 16)] - b * N
                        rowv = lax.shift_right_logical(nloc, 3)
                        colb = (nloc & 7) * FE
                        ocol = iota16 + g * 16
                        for col in range(FE):
                            v = plsc.load_gather(
                                xb, [rowv, colb + col])
                            plsc.store_scatter(
                                chb, [jnp.full((16,), col, jnp.int32), ocol],
                                v)
                    base = k * BM + b * M + c * CH
                    pltpu.sync_copy(chb, o_hbm.at[:, pl.ds(base, CH)])

    return gk(x128, gidxt)


def _arccos(v):
    a = jnp.abs(v)
    p = jnp.float32(-0.0012624911)
    p = p * a + jnp.float32(0.0066700901)
    p = p * a + jnp.float32(-0.0170881256)
    p = p * a + jnp.float32(0.0308918810)
    p = p * a + jnp.float32(-0.0501743046)
    p = p * a + jnp.float32(0.0889789874)
    p = p * a + jnp.float32(-0.2145988016)
    p = p * a + jnp.float32(1.5707963050)
    r = jnp.sqrt(jnp.maximum(1.0 - a, 0.0)) * p
    return jnp.where(v < 0, jnp.float32(3.1415927410125732) - r, r)


def _mlp_body(xt_ref, valst_ref, w1at_ref, w1dt_ref, b1t_ref, w2t_ref,
              b2_ref, out_ref, lgt_ref):
    dT = _arccos(jnp.clip(valst_ref[...], -1.0, 1.0))
    w1at = w1at_ref[...]
    w1dt = w1dt_ref[...]
    b1t = b1t_ref[...]
    w2t = w2t_ref[...]
    for k in range(K):
        xk = xt_ref[:, k, :]
        a1k = jnp.dot(w1at, xk, preferred_element_type=jnp.float32)
        hk = jnp.maximum(a1k + w1dt * dT[k:k + 1, :] + b1t, 0.0)
        lgt_ref[k:k + 1, :] = jnp.sum(hk * w2t, axis=0, keepdims=True)
    lgt = lgt_ref[...] + b2_ref[0, 0]
    mx = jnp.max(lgt, axis=0, keepdims=True)
    e = jnp.exp(lgt - mx)
    wT = e / jnp.sum(e, axis=0, keepdims=True)
    poolT = jnp.zeros((FE, TR), jnp.float32)
    for k in range(K):
        poolT = poolT + wT[k:k + 1, :] * xt_ref[:, k, :]
    out_ref[...] = jnp.transpose(poolT, (1, 0))


def _run_mlp(xt3, valst, W1aT, w1dT, b1T, W2c, b2, interpret=False):
    grid = (BM // TR,)
    return pl.pallas_call(
        _mlp_body,
        grid=grid,
        in_specs=[
            pl.BlockSpec((FE, K, TR), lambda r: (0, 0, r)),
            pl.BlockSpec((K, TR), lambda r: (0, r)),
            pl.BlockSpec((H, FE), lambda r: (0, 0)),
            pl.BlockSpec((H, 1), lambda r: (0, 0)),
            pl.BlockSpec((H, 1), lambda r: (0, 0)),
            pl.BlockSpec((H, 1), lambda r: (0, 0)),
            pl.BlockSpec((1, 1), lambda r: (0, 0)),
        ],
        out_specs=pl.BlockSpec((TR, FE), lambda r: (r, 0)),
        out_shape=jax.ShapeDtypeStruct((BM, FE), jnp.float32),
        scratch_shapes=[pltpu.VMEM((K, TR), jnp.float32)],
        interpret=interpret,
    )(xt3, valst, W1aT, w1dT, b1T, W2c, b2)


def _sph_to_cart(theta, phi):
    st = jnp.sin(theta)
    return jnp.stack([st * jnp.cos(phi), st * jnp.sin(phi),
                      jnp.cos(theta)], axis=-1)


def kernel(x, los_theta_phi, pix_theta_phi, W1, b1, W2, b2):
    r_los = _sph_to_cart(los_theta_phi[..., 0], los_theta_phi[..., 1])
    r_pix = _sph_to_cart(pix_theta_phi[:, 0], pix_theta_phi[:, 1])
    r_losT = jnp.transpose(r_los, (0, 2, 1))

    valst, idx, gidxt = _run_topk(r_pix, r_losT)

    xgT = _sc_gather2(x.reshape(B * N * FE // 128, 128),
                      gidxt.reshape(K * BM))
    xt3 = xgT.reshape(FE, K, BM)

    W1aT = jnp.transpose(W1[:FE, :], (1, 0))
    w1dT = jnp.transpose(W1[FE:FE + 1, :], (1, 0))
    pooled = _run_mlp(xt3, valst, W1aT, w1dT,
                      b1.reshape(H, 1), W2, b2.reshape(1, 1))
    return pooled.reshape(B, M, FE), idx
